# Initial kernel scaffold; baseline (speedup 1.0000x reference)
import jax
import jax.numpy as jnp
from jax.experimental import pallas as pl


def kernel(hidden, logits, db_keys, db_token_ids):
    raise NotImplementedError("write your pallas kernel here")


# speedup vs baseline: 1.8653x; 1.8653x over previous
import functools

import jax
import jax.numpy as jnp
from jax import lax
from jax.experimental import pallas as pl
from jax.experimental.pallas import tpu as pltpu

_TOP_K = 8
_MIX = 0.25
_BW = 10.0
_CHUNK = 2048


def _topk_body(k_total, qT_ref, keys_ref, best_d_ref, best_i_ref):
    i = pl.program_id(0)
    c = keys_ref.shape[0]
    k = keys_ref[...]
    scores = jnp.dot(k, qT_ref[...], preferred_element_type=jnp.float32)
    ksq = jnp.sum(k * k, axis=1, keepdims=True)
    d = ksq - 2.0 * scores
    row = lax.broadcasted_iota(jnp.int32, d.shape, 0) + i * c
    d = jnp.where(row < k_total, d, jnp.inf)
    imax = jnp.iinfo(jnp.int32).max

    cd, ci = [], []
    for _ in range(_TOP_K):
        m = jnp.min(d, axis=0, keepdims=True)
        am = jnp.min(jnp.where(d == m, row, imax), axis=0, keepdims=True)
        cd.append(m)
        ci.append(am)
        d = jnp.where(row == am, jnp.inf, d)
    cdm = jnp.concatenate(cd, axis=0)
    cim = jnp.concatenate(ci, axis=0)

    @pl.when(i == 0)
    def _():
        best_d_ref[...] = cdm
        best_i_ref[...] = cim

    @pl.when(i > 0)
    def _():
        wd = jnp.concatenate([best_d_ref[...], cdm], axis=0)
        wi = jnp.concatenate([best_i_ref[...], cim], axis=0)
        nd, ni = [], []
        for _ in range(_TOP_K):
            m = jnp.min(wd, axis=0, keepdims=True)
            am = jnp.min(jnp.where(wd == m, wi, imax), axis=0, keepdims=True)
            nd.append(m)
            ni.append(am)
            wd = jnp.where(wi == am, jnp.inf, wd)
        best_d_ref[...] = jnp.concatenate(nd, axis=0)
        best_i_ref[...] = jnp.concatenate(ni, axis=0)


def _topk(qT, db_keys, interpret=False):
    h, q = qT.shape
    k_total = db_keys.shape[0]
    grid = (k_total + _CHUNK - 1) // _CHUNK
    return pl.pallas_call(
        functools.partial(_topk_body, k_total),
        grid=(grid,),
        in_specs=[
            pl.BlockSpec((h, q), lambda i: (0, 0)),
            pl.BlockSpec((_CHUNK, h), lambda i: (i, 0)),
        ],
        out_specs=[
            pl.BlockSpec((_TOP_K, q), lambda i: (0, 0)),
            pl.BlockSpec((_TOP_K, q), lambda i: (0, 0)),
        ],
        out_shape=[
            jax.ShapeDtypeStruct((_TOP_K, q), jnp.float32),
            jax.ShapeDtypeStruct((_TOP_K, q), jnp.int32),
        ],
        interpret=interpret,
    )(qT, db_keys)


def _weights_body(d_ref, flat_ref, w_ref):
    d = d_ref[...]
    m = jnp.min(d, axis=0, keepdims=True)
    e = jnp.exp((m - d) / _BW)
    w = e / jnp.sum(e, axis=0, keepdims=True)
    f = flat_ref[...]
    wt = jnp.zeros_like(w)
    for c in range(_TOP_K):
        wt = wt + jnp.where(f == f[c:c + 1, :], w[c:c + 1, :], 0.0)
    w_ref[...] = wt


def _weights(best_d, flat8, interpret=False):
    q = best_d.shape[1]
    return pl.pallas_call(
        _weights_body,
        out_shape=jax.ShapeDtypeStruct((_TOP_K, q), jnp.float32),
        interpret=interpret,
    )(best_d, flat8)


def _mix_body(lg_ref, acc_ref, out_ref):
    lg = lg_ref[...]
    m = jnp.max(lg, axis=1, keepdims=True)
    e = jnp.exp(lg - m)
    sm = e / jnp.sum(e, axis=1, keepdims=True)
    out_ref[...] = jnp.log((1.0 - _MIX) * sm + _MIX * acc_ref[...])


def _mix(lg, acc, interpret=False):
    q, v = lg.shape
    rb = 16
    return pl.pallas_call(
        _mix_body,
        grid=(q // rb,),
        in_specs=[
            pl.BlockSpec((rb, v), lambda i: (i, 0)),
            pl.BlockSpec((rb, v), lambda i: (i, 0)),
        ],
        out_specs=pl.BlockSpec((rb, v), lambda i: (i, 0)),
        out_shape=jax.ShapeDtypeStruct((q, v), jnp.float32),
        interpret=interpret,
    )(lg, acc)


def kernel(hidden, logits, db_keys, db_token_ids):
    b, s, h = hidden.shape
    v = logits.shape[-1]
    q = b * s
    qm = hidden.reshape(q, h)
    lg = logits.reshape(q, v)

    best_d, best_i = _topk(qm.T, db_keys)

    bi_flat = best_i.T.reshape(-1)
    tok = db_token_ids[bi_flat]
    qidx = jnp.arange(q * _TOP_K, dtype=jnp.int32) // _TOP_K
    flat = qidx * v + tok

    flat8 = flat.reshape(q, _TOP_K).T
    w_tot = _weights(best_d, flat8)
    val = w_tot.T.reshape(-1)

    acc = jnp.zeros((q * v,), jnp.float32).at[flat].set(val)

    out = _mix(lg, acc.reshape(q, v))
    return out.reshape(b, s, v)
